# Initial kernel scaffold; baseline (speedup 1.0000x reference)
import functools

import jax
import jax.numpy as jnp
from jax import lax
from jax.experimental import pallas as pl
from jax.experimental.pallas import tpu as pltpu
from jax.experimental.pallas import tpu_sc as plsc

LN2 = 0.6931471805599453
NEG = -3.0e38
NT = 32
ECH = 6400


def _sp(v):
    return jnp.maximum(v, 0.0) + jnp.log(1.0 + jnp.exp(-jnp.abs(v)))


def _sc_mesh():
    return plsc.VectorSubcoreMesh(core_axis_name="c", subcore_axis_name="s",
                                  num_cores=2, num_subcores=16)


def _tc0_body(x_ref, ea_ref, ei_ref, embA_ref, embB_ref, Wb_ref,
              aT_ref, A1T_ref, Bm1T_ref, packed_ref):
    nA = embA_ref.shape[0]
    nB = embB_ref.shape[0]
    n = x_ref.shape[1]
    d = embA_ref.shape[1]
    ohA = (lax.broadcasted_iota(jnp.int32, (nA, n), 0) == x_ref[...]
           ).astype(jnp.float32)
    aT_ref[...] = _sp(lax.dot_general(
        embA_ref[...], ohA, (((0,), (0,)), ((), ())),
        preferred_element_type=jnp.float32))
    ohB = (lax.broadcasted_iota(jnp.int32, (nB, n), 0) == ea_ref[...]
           ).astype(jnp.float32)
    bT = _sp(lax.dot_general(
        embB_ref[...], ohB, (((0,), (0,)), ((), ())),
        preferred_element_type=jnp.float32))
    Wb = Wb_ref[...]
    W2b = Wb[d:, :]
    Wd = Wb[:d, :] - W2b
    A1T_ref[...] = lax.dot_general(Wd, bT, (((0,), (0,)), ((), ())),
                                   preferred_element_type=jnp.float32)
    Bm1T_ref[...] = lax.dot_general(W2b, bT, (((0,), (0,)), ((), ())),
                                    preferred_element_type=jnp.float32)
    packed_ref[...] = ei_ref[1:2, :] * 65536 + ei_ref[0:1, :]


def _tc_nb_body(with_next, AT_ref, maxT_ref, deg_ref, bb_ref, Wb_ref,
                nbT_ref, *next_refs):
    d = AT_ref.shape[0]
    nb = jnp.where(deg_ref[...] > 0.0,
                   _sp(AT_ref[...] + maxT_ref[...] + bb_ref[...]),
                   LN2)
    nbT_ref[...] = nb
    if with_next:
        A2T_ref, Bm2T_ref = next_refs
        Wb = Wb_ref[...]
        W2b = Wb[d:, :]
        Wd = Wb[:d, :] - W2b
        A2T_ref[...] = lax.dot_general(Wd, nb, (((0,), (0,)), ((), ())),
                                       preferred_element_type=jnp.float32)
        Bm2T_ref[...] = lax.dot_general(W2b, nb, (((0,), (0,)), ((), ())),
                                        preferred_element_type=jnp.float32)


def _agg2(sumAT, sumbT, deg, degs, Wm, We, bsum):
    sbf = sumbT + LN2 * (deg - degs)
    return (lax.dot_general(Wm, sumAT, (((0,), (0,)), ((), ())),
                            preferred_element_type=jnp.float32)
            + lax.dot_general(We, sbf, (((0,), (0,)), ((), ())),
                              preferred_element_type=jnp.float32)
            + deg * bsum)


def _tc2_body(sumAT_ref, sumbT_ref, deg_ref, degs_ref, aprevT_ref,
              Wm_ref, We_ref, bsum_ref, aT_ref):
    agg = _agg2(sumAT_ref[...], sumbT_ref[...], deg_ref[...], degs_ref[...],
                Wm_ref[...], We_ref[...], bsum_ref[...])
    aT_ref[...] = _sp(agg + aprevT_ref[...])


def _tc3_body(AT_ref, maxT_ref, deg_ref, bb_ref, fill_ref,
              nbT_ref, ob_ref):
    nb = jnp.where(deg_ref[...] > 0.0,
                   _sp(AT_ref[...] + maxT_ref[...] + bb_ref[...]),
                   LN2)
    nbT_ref[...] = nb
    ob_ref[...] = jnp.transpose(nb, (1, 0))


def _tc4_body(sumAT_ref, sumbT_ref, deg_ref, degs_ref, aprevT_ref,
              Wm_ref, We_ref, bsum_ref, batch_ref,
              W1_ref, b1_ref, W2_ref, b2_ref, W3_ref, b3_ref,
              outT_ref, atom_ref):
    n = sumAT_ref.shape[1]
    ng = 256
    agg = _agg2(sumAT_ref[...], sumbT_ref[...], deg_ref[...], degs_ref[...],
                Wm_ref[...], We_ref[...], bsum_ref[...])
    a2 = _sp(agg + aprevT_ref[...])
    atom_ref[...] = jnp.transpose(a2, (1, 0))
    ohg = (lax.broadcasted_iota(jnp.int32, (ng, n), 0) == batch_ref[...]
           ).astype(jnp.float32)
    pooledT = lax.dot_general(a2, ohg, (((1,), (1,)), ((), ())),
                              preferred_element_type=jnp.float32)
    h = _sp(lax.dot_general(W1_ref[...], pooledT, (((0,), (0,)), ((), ())),
                            preferred_element_type=jnp.float32)
            + b1_ref[...])
    h = _sp(lax.dot_general(W2_ref[...], h, (((0,), (0,)), ((), ())),
                            preferred_element_type=jnp.float32)
            + b2_ref[...])
    outT_ref[...] = lax.dot_general(W3_ref[...], h, (((0,), (0,)), ((), ())),
                                    preferred_element_type=jnp.float32) \
        + b3_ref[...]


def _fill_body(ob_ref):
    ob_ref[...] = jnp.full(ob_ref.shape, LN2, jnp.float32)


def _make_edge_pass(n_nodes, n_edges, d, with_deg):
    cpt = d // NT
    outs = [jax.ShapeDtypeStruct((d, n_nodes), jnp.float32),
            jax.ShapeDtypeStruct((d, n_nodes), jnp.float32)]
    scratch = [pltpu.VMEM((cpt, n_nodes), jnp.float32),
               pltpu.VMEM((cpt, n_nodes), jnp.float32),
               pltpu.VMEM((ECH,), jnp.int32)]
    if with_deg:
        outs.append(jax.ShapeDtypeStruct((n_nodes,), jnp.float32))
        scratch.append(pltpu.VMEM((n_nodes,), jnp.float32))

    @functools.partial(pl.kernel, out_type=tuple(outs), mesh=_sc_mesh(),
                       scratch_types=scratch)
    def edge_pass(packed, bmT, aT, maxT, sumT, *rest):
        if with_deg:
            degO, tin, tout, ebuf, degv = rest
        else:
            tin, tout, ebuf = rest
        cid = lax.axis_index("c")
        sid = lax.axis_index("s")
        wid = sid * 2 + cid
        base = wid * cpt
        ccs = [jnp.full((16,), c, jnp.int32) for c in range(cpt)]
        ones16 = jnp.full((16,), 1.0, jnp.float32)

        def zero_tbl(tbl, val):
            def zb(j, _):
                for c in range(cpt):
                    tbl[c, pl.ds(j * 16, 16)] = jnp.full((16,), val,
                                                         jnp.float32)
                return 0
            lax.fori_loop(0, n_nodes // 16, zb, 0)

        def edge_loop(per_group):
            def chunk_body(ch, _):
                pltpu.sync_copy(packed.at[pl.ds(ch * ECH, ECH)], ebuf)

                def grp(g, _):
                    p = ebuf[pl.ds(g * 16, 16)]
                    dct = lax.shift_right_logical(p, 16)
                    srcv = lax.bitwise_and(p, 65535)
                    per_group(dct, srcv)
                    return 0
                lax.fori_loop(0, ECH // 16, grp, 0)
                return 0
            lax.fori_loop(0, n_edges // ECH, chunk_body, 0)

        pltpu.sync_copy(bmT.at[pl.ds(base, cpt), :], tin)
        zero_tbl(tout, NEG)
        if with_deg:
            def zd(j, _):
                degv[pl.ds(j * 16, 16)] = jnp.zeros((16,), jnp.float32)
                return 0
            lax.fori_loop(0, n_nodes // 16, zd, 0)

        def max_group(dct, srcv):
            gs = [plsc.load_gather(tin, [ccs[c], srcv]) for c in range(cpt)]
            if with_deg:
                plsc.addupdate_scatter(degv, [dct], ones16)
            cur = tuple(plsc.load_gather(tout, [ccs[c], dct])
                        for c in range(cpt))

            def cond(cu):
                acc = jnp.any(gs[0] > cu[0])
                for c in range(1, cpt):
                    acc = jnp.logical_or(acc, jnp.any(gs[c] > cu[c]))
                return acc

            def bodyw(cu):
                for c in range(cpt):
                    plsc.store_scatter(tout, [ccs[c], dct], gs[c],
                                       mask=gs[c] > cu[c])
                return tuple(plsc.load_gather(tout, [ccs[c], dct])
                             for c in range(cpt))
            lax.while_loop(cond, bodyw, cur)

        edge_loop(max_group)
        pltpu.sync_copy(tout, maxT.at[pl.ds(base, cpt), :])
        if with_deg:
            @pl.when(wid == 0)
            def _():
                pltpu.sync_copy(degv, degO)

        pltpu.sync_copy(aT.at[pl.ds(base, cpt), :], tin)
        zero_tbl(tout, 0.0)

        def sum_group(dct, srcv):
            for c in range(cpt):
                gv = plsc.load_gather(tin, [ccs[c], srcv])
                plsc.addupdate_scatter(tout, [ccs[c], dct], gv)

        edge_loop(sum_group)
        pltpu.sync_copy(tout, sumT.at[pl.ds(base, cpt), :])

    return edge_pass


def _make_head_pass(n_nodes, d, with_deg):
    cpt = d // NT
    outs = [jax.ShapeDtypeStruct((d, n_nodes), jnp.float32)]
    scratch = [pltpu.VMEM((cpt, n_nodes), jnp.float32),
               pltpu.VMEM((cpt, n_nodes), jnp.float32),
               pltpu.VMEM((n_nodes,), jnp.int32)]
    if with_deg:
        outs.append(jax.ShapeDtypeStruct((n_nodes,), jnp.float32))
        scratch.append(pltpu.VMEM((n_nodes,), jnp.float32))

    @functools.partial(pl.kernel, out_type=tuple(outs), mesh=_sc_mesh(),
                       scratch_types=scratch)
    def head_pass(packed, nbT, sumT, *rest):
        if with_deg:
            degO, tin, tout, ebuf, degv = rest
        else:
            tin, tout, ebuf = rest
        cid = lax.axis_index("c")
        sid = lax.axis_index("s")
        wid = sid * 2 + cid
        base = wid * cpt
        ccs = [jnp.full((16,), c, jnp.int32) for c in range(cpt)]
        ones16 = jnp.full((16,), 1.0, jnp.float32)

        pltpu.sync_copy(nbT.at[pl.ds(base, cpt), :], tin)
        pltpu.sync_copy(packed.at[pl.ds(0, n_nodes)], ebuf)

        def zb(j, _):
            for c in range(cpt):
                tout[c, pl.ds(j * 16, 16)] = jnp.zeros((16,), jnp.float32)
            if with_deg:
                degv[pl.ds(j * 16, 16)] = jnp.zeros((16,), jnp.float32)
            return 0
        lax.fori_loop(0, n_nodes // 16, zb, 0)

        def grp(g, _):
            p = ebuf[pl.ds(g * 16, 16)]
            dct = lax.shift_right_logical(p, 16)
            if with_deg:
                plsc.addupdate_scatter(degv, [dct], ones16)
            for c in range(cpt):
                gv = tin[c, pl.ds(g * 16, 16)]
                plsc.addupdate_scatter(tout, [ccs[c], dct], gv)
            return 0
        lax.fori_loop(0, n_nodes // 16, grp, 0)

        pltpu.sync_copy(tout, sumT.at[pl.ds(base, cpt), :])
        if with_deg:
            @pl.when(wid == 0)
            def _():
                pltpu.sync_copy(degv, degO)

    return head_pass


def kernel(x, edge_attr, edge_index, batch, emb_atom_w, emb_bond_w,
           W_msg, b_msg, W_edge, b_edge, W_bond, b_bond,
           W1, b1, W2, b2, W3, b3):
    n = x.shape[0]
    e = edge_attr.shape[0]
    d = emb_atom_w.shape[1]
    rd = W1.shape[1]
    f32 = jnp.float32
    nodeT = jax.ShapeDtypeStruct((d, n), f32)

    x2 = x.reshape(1, n).astype(jnp.int32)
    ea2 = edge_attr[:n].reshape(1, n).astype(jnp.int32)
    ei = edge_index.astype(jnp.int32)
    batch2 = batch.reshape(1, n).astype(jnp.int32)
    bb_col = b_bond.reshape(d, 1)
    bsum_col = (b_msg + b_edge).reshape(d, 1)

    aT0, A1T, Bm1T, packed2 = pl.pallas_call(
        _tc0_body,
        out_shape=(nodeT, nodeT, nodeT,
                   jax.ShapeDtypeStruct((1, e), jnp.int32)),
    )(x2, ea2, ei, emb_atom_w, emb_bond_w, W_bond)
    packed = packed2.reshape(e)

    fill = pl.pallas_call(
        _fill_body,
        grid=(32,),
        out_shape=jax.ShapeDtypeStruct((e, d), f32),
        out_specs=pl.BlockSpec((e // 32, d), lambda i: (i, 0)),
    )()

    edge_pass_deg = _make_edge_pass(n, e, d, True)
    edge_pass = _make_edge_pass(n, e, d, False)
    head_pass_deg = _make_head_pass(n, d, True)
    head_pass = _make_head_pass(n, d, False)

    maxB1T, sumA1T, deg = edge_pass_deg(packed, Bm1T, aT0)
    deg_row = deg.reshape(1, n)
    nb1T, A2T, Bm2T = pl.pallas_call(
        functools.partial(_tc_nb_body, True),
        out_shape=(nodeT, nodeT, nodeT),
    )(A1T, maxB1T, deg_row, bb_col, W_bond)
    sumb1T, degs = head_pass_deg(packed, nb1T)
    degs_row = degs.reshape(1, n)
    a1T = pl.pallas_call(
        _tc2_body,
        out_shape=nodeT,
    )(sumA1T, sumb1T, deg_row, degs_row, aT0, W_msg, W_edge, bsum_col)

    maxB2T, sumA2T = edge_pass(packed, Bm2T, a1T)
    nb2T, out_bond = pl.pallas_call(
        _tc3_body,
        grid=(1,),
        out_shape=(nodeT, jax.ShapeDtypeStruct((e, d), f32)),
        in_specs=[pl.BlockSpec((d, n), lambda i: (0, 0)),
                  pl.BlockSpec((d, n), lambda i: (0, 0)),
                  pl.BlockSpec((1, n), lambda i: (0, 0)),
                  pl.BlockSpec((d, 1), lambda i: (0, 0)),
                  pl.BlockSpec((n, d), lambda i: (0, 0))],
        out_specs=(pl.BlockSpec((d, n), lambda i: (0, 0)),
                   pl.BlockSpec((n, d), lambda i: (0, 0))),
        input_output_aliases={4: 1},
    )(A2T, maxB2T, deg_row, bb_col, fill)
    sumb2T, = head_pass(packed, nb2T)

    outT, out_atom = pl.pallas_call(
        _tc4_body,
        out_shape=(jax.ShapeDtypeStruct((1, 256), f32),
                   jax.ShapeDtypeStruct((n, d), f32)),
    )(sumA2T, sumb2T, deg_row, degs_row, a1T, W_msg, W_edge, bsum_col,
      batch2, W1, b1.reshape(rd, 1), W2, b2.reshape(rd, 1), W3,
      b3.reshape(1, 1))
    out = outT.reshape(256, 1)
    return (out, out_atom, out_bond)


# speedup vs baseline: 1.3874x; 1.3874x over previous
import functools

import jax
import jax.numpy as jnp
from jax import lax
from jax.experimental import pallas as pl
from jax.experimental.pallas import tpu as pltpu
from jax.experimental.pallas import tpu_sc as plsc

LN2 = 0.6931471805599453
NEG = -3.0e38
NT = 32
ECH = 6400


def _sp(v):
    return jnp.maximum(v, 0.0) + jnp.log(1.0 + jnp.exp(-jnp.abs(v)))


def _sc_mesh():
    return plsc.VectorSubcoreMesh(core_axis_name="c", subcore_axis_name="s",
                                  num_cores=2, num_subcores=16)


def _tc0_body(x_ref, ea_ref, ei_ref, embA_ref, embB_ref, Wb_ref,
              aT_ref, A1T_ref, Bm1T_ref, packed_ref):
    nA = embA_ref.shape[0]
    nB = embB_ref.shape[0]
    n = x_ref.shape[1]
    d = embA_ref.shape[1]
    ohA = (lax.broadcasted_iota(jnp.int32, (nA, n), 0) == x_ref[...]
           ).astype(jnp.float32)
    aT_ref[...] = _sp(lax.dot_general(
        embA_ref[...], ohA, (((0,), (0,)), ((), ())),
        preferred_element_type=jnp.float32,
        precision=lax.Precision.HIGHEST))
    ohB = (lax.broadcasted_iota(jnp.int32, (nB, n), 0) == ea_ref[...]
           ).astype(jnp.float32)
    bT = _sp(lax.dot_general(
        embB_ref[...], ohB, (((0,), (0,)), ((), ())),
        preferred_element_type=jnp.float32,
        precision=lax.Precision.HIGHEST))
    Wb = Wb_ref[...]
    W2b = Wb[d:, :]
    Wd = Wb[:d, :] - W2b
    A1T_ref[...] = lax.dot_general(Wd, bT, (((0,), (0,)), ((), ())),
                                   preferred_element_type=jnp.float32,
        precision=lax.Precision.HIGHEST)
    Bm1T_ref[...] = lax.dot_general(W2b, bT, (((0,), (0,)), ((), ())),
                                    preferred_element_type=jnp.float32,
        precision=lax.Precision.HIGHEST)
    packed_ref[...] = ei_ref[1:2, :] * 65536 + ei_ref[0:1, :]


def _tc_nb_body(with_next, AT_ref, maxT_ref, deg_ref, bb_ref, Wb_ref,
                nbT_ref, *next_refs):
    d = AT_ref.shape[0]
    nb = jnp.where(deg_ref[...] > 0.0,
                   _sp(AT_ref[...] + maxT_ref[...] + bb_ref[...]),
                   LN2)
    nbT_ref[...] = nb
    if with_next:
        A2T_ref, Bm2T_ref = next_refs
        Wb = Wb_ref[...]
        W2b = Wb[d:, :]
        Wd = Wb[:d, :] - W2b
        A2T_ref[...] = lax.dot_general(Wd, nb, (((0,), (0,)), ((), ())),
                                       preferred_element_type=jnp.float32,
        precision=lax.Precision.HIGHEST)
        Bm2T_ref[...] = lax.dot_general(W2b, nb, (((0,), (0,)), ((), ())),
                                        preferred_element_type=jnp.float32,
        precision=lax.Precision.HIGHEST)


def _agg2(sumAT, sumbT, deg, degs, Wm, We, bsum):
    sbf = sumbT + LN2 * (deg - degs)
    return (lax.dot_general(Wm, sumAT, (((0,), (0,)), ((), ())),
                            preferred_element_type=jnp.float32,
        precision=lax.Precision.HIGHEST)
            + lax.dot_general(We, sbf, (((0,), (0,)), ((), ())),
                              preferred_element_type=jnp.float32,
        precision=lax.Precision.HIGHEST)
            + deg * bsum)


def _tc2_body(sumAT_ref, sumbT_ref, deg_ref, degs_ref, aprevT_ref,
              Wm_ref, We_ref, bsum_ref, aT_ref):
    agg = _agg2(sumAT_ref[...], sumbT_ref[...], deg_ref[...], degs_ref[...],
                Wm_ref[...], We_ref[...], bsum_ref[...])
    aT_ref[...] = _sp(agg + aprevT_ref[...])


def _tc3_body(AT_ref, maxT_ref, deg_ref, bb_ref, fill_ref,
              nbT_ref, ob_ref):
    nb = jnp.where(deg_ref[...] > 0.0,
                   _sp(AT_ref[...] + maxT_ref[...] + bb_ref[...]),
                   LN2)
    nbT_ref[...] = nb
    ob_ref[...] = jnp.transpose(nb, (1, 0))


def _tc4_body(sumAT_ref, sumbT_ref, deg_ref, degs_ref, aprevT_ref,
              Wm_ref, We_ref, bsum_ref, batch_ref,
              W1_ref, b1_ref, W2_ref, b2_ref, W3_ref, b3_ref,
              outT_ref, atom_ref):
    n = sumAT_ref.shape[1]
    ng = 256
    agg = _agg2(sumAT_ref[...], sumbT_ref[...], deg_ref[...], degs_ref[...],
                Wm_ref[...], We_ref[...], bsum_ref[...])
    a2 = _sp(agg + aprevT_ref[...])
    atom_ref[...] = jnp.transpose(a2, (1, 0))
    ohg = (lax.broadcasted_iota(jnp.int32, (ng, n), 0) == batch_ref[...]
           ).astype(jnp.float32)
    pooledT = lax.dot_general(a2, ohg, (((1,), (1,)), ((), ())),
                              preferred_element_type=jnp.float32,
        precision=lax.Precision.HIGHEST)
    h = _sp(lax.dot_general(W1_ref[...], pooledT, (((0,), (0,)), ((), ())),
                            preferred_element_type=jnp.float32,
        precision=lax.Precision.HIGHEST)
            + b1_ref[...])
    h = _sp(lax.dot_general(W2_ref[...], h, (((0,), (0,)), ((), ())),
                            preferred_element_type=jnp.float32,
        precision=lax.Precision.HIGHEST)
            + b2_ref[...])
    outT_ref[...] = lax.dot_general(W3_ref[...], h, (((0,), (0,)), ((), ())),
                                    preferred_element_type=jnp.float32,
        precision=lax.Precision.HIGHEST) \
        + b3_ref[...]


def _fill_body(ob_ref):
    ob_ref[...] = jnp.full(ob_ref.shape, LN2, jnp.float32)


def _make_edge_pass(n_nodes, n_edges, d, with_deg):
    cpt = d // NT
    outs = [jax.ShapeDtypeStruct((d * n_nodes,), jnp.float32),
            jax.ShapeDtypeStruct((d * n_nodes,), jnp.float32)]
    scratch = [pltpu.VMEM((cpt * n_nodes,), jnp.float32),
               pltpu.VMEM((cpt * n_nodes,), jnp.float32),
               pltpu.VMEM((ECH,), jnp.int32)]
    if with_deg:
        outs.append(jax.ShapeDtypeStruct((n_nodes,), jnp.float32))
        scratch.append(pltpu.VMEM((n_nodes,), jnp.float32))

    @functools.partial(
        pl.kernel, out_type=tuple(outs), mesh=_sc_mesh(),
        scratch_types=scratch,
        compiler_params=pltpu.CompilerParams(needs_layout_passes=False))
    def edge_pass(packed, bmT, aT, maxT, sumT, *rest):
        if with_deg:
            degO, tin, tout, ebuf, degv = rest
        else:
            tin, tout, ebuf = rest
        cid = lax.axis_index("c")
        sid = lax.axis_index("s")
        wid = sid * 2 + cid
        base = wid * cpt * n_nodes
        offs = [jnp.full((16,), c * n_nodes, jnp.int32) for c in range(cpt)]
        ones16 = jnp.full((16,), 1.0, jnp.float32)

        def zero_tbl(tbl, val):
            def zb(j, _):
                tbl[pl.ds(j * 16, 16)] = jnp.full((16,), val, jnp.float32)
                return 0
            lax.fori_loop(0, cpt * n_nodes // 16, zb, 0)

        def edge_loop(per_group):
            def chunk_body(ch, _):
                pltpu.sync_copy(packed.at[pl.ds(ch * ECH, ECH)], ebuf)

                def grp(g, _):
                    p = ebuf[pl.ds(g * 16, 16)]
                    dct = lax.shift_right_logical(p, 16)
                    srcv = lax.bitwise_and(p, 65535)
                    per_group(dct, srcv)
                    return 0
                lax.fori_loop(0, ECH // 16, grp, 0)
                return 0
            lax.fori_loop(0, n_edges // ECH, chunk_body, 0)

        pltpu.sync_copy(bmT.at[pl.ds(base, cpt * n_nodes)], tin)
        zero_tbl(tout, NEG)
        if with_deg:
            def zd(j, _):
                degv[pl.ds(j * 16, 16)] = jnp.zeros((16,), jnp.float32)
                return 0
            lax.fori_loop(0, n_nodes // 16, zd, 0)

        def max_group(dct, srcv):
            gs = [plsc.load_gather(tin, [offs[c] + srcv]) for c in range(cpt)]
            if with_deg:
                plsc.addupdate_scatter(degv, [dct], ones16)
            dis = [offs[c] + dct for c in range(cpt)]
            cur = tuple(plsc.load_gather(tout, [dis[c]])
                        for c in range(cpt))

            def cond(cu):
                acc = jnp.any(gs[0] > cu[0])
                for c in range(1, cpt):
                    acc = jnp.logical_or(acc, jnp.any(gs[c] > cu[c]))
                return acc

            def bodyw(cu):
                for c in range(cpt):
                    plsc.store_scatter(tout, [dis[c]], gs[c],
                                       mask=gs[c] > cu[c])
                return tuple(plsc.load_gather(tout, [dis[c]])
                             for c in range(cpt))
            lax.while_loop(cond, bodyw, cur)

        edge_loop(max_group)
        pltpu.sync_copy(tout, maxT.at[pl.ds(base, cpt * n_nodes)])
        if with_deg:
            @pl.when(wid == 0)
            def _():
                pltpu.sync_copy(degv, degO)

        pltpu.sync_copy(aT.at[pl.ds(base, cpt * n_nodes)], tin)
        zero_tbl(tout, 0.0)

        def sum_group(dct, srcv):
            for c in range(cpt):
                gv = plsc.load_gather(tin, [offs[c] + srcv])
                plsc.addupdate_scatter(tout, [offs[c] + dct], gv)

        edge_loop(sum_group)
        pltpu.sync_copy(tout, sumT.at[pl.ds(base, cpt * n_nodes)])

    return edge_pass


def _make_head_pass(n_nodes, d, with_deg):
    cpt = d // NT
    outs = [jax.ShapeDtypeStruct((d * n_nodes,), jnp.float32)]
    scratch = [pltpu.VMEM((cpt * n_nodes,), jnp.float32),
               pltpu.VMEM((cpt * n_nodes,), jnp.float32),
               pltpu.VMEM((n_nodes,), jnp.int32)]
    if with_deg:
        outs.append(jax.ShapeDtypeStruct((n_nodes,), jnp.float32))
        scratch.append(pltpu.VMEM((n_nodes,), jnp.float32))

    @functools.partial(
        pl.kernel, out_type=tuple(outs), mesh=_sc_mesh(),
        scratch_types=scratch,
        compiler_params=pltpu.CompilerParams(needs_layout_passes=False))
    def head_pass(packed, nbT, sumT, *rest):
        if with_deg:
            degO, tin, tout, ebuf, degv = rest
        else:
            tin, tout, ebuf = rest
        cid = lax.axis_index("c")
        sid = lax.axis_index("s")
        wid = sid * 2 + cid
        base = wid * cpt * n_nodes
        offs = [jnp.full((16,), c * n_nodes, jnp.int32) for c in range(cpt)]
        ones16 = jnp.full((16,), 1.0, jnp.float32)

        pltpu.sync_copy(nbT.at[pl.ds(base, cpt * n_nodes)], tin)
        pltpu.sync_copy(packed.at[pl.ds(0, n_nodes)], ebuf)

        def zb(j, _):
            tout[pl.ds(j * 16, 16)] = jnp.zeros((16,), jnp.float32)
            return 0
        lax.fori_loop(0, cpt * n_nodes // 16, zb, 0)
        if with_deg:
            def zd(j, _):
                degv[pl.ds(j * 16, 16)] = jnp.zeros((16,), jnp.float32)
                return 0
            lax.fori_loop(0, n_nodes // 16, zd, 0)

        def grp(g, _):
            p = ebuf[pl.ds(g * 16, 16)]
            dct = lax.shift_right_logical(p, 16)
            if with_deg:
                plsc.addupdate_scatter(degv, [dct], ones16)
            for c in range(cpt):
                gv = tin[pl.ds(c * n_nodes + g * 16, 16)]
                plsc.addupdate_scatter(tout, [offs[c] + dct], gv)
            return 0
        lax.fori_loop(0, n_nodes // 16, grp, 0)

        pltpu.sync_copy(tout, sumT.at[pl.ds(base, cpt * n_nodes)])
        if with_deg:
            @pl.when(wid == 0)
            def _():
                pltpu.sync_copy(degv, degO)

    return head_pass


def kernel(x, edge_attr, edge_index, batch, emb_atom_w, emb_bond_w,
           W_msg, b_msg, W_edge, b_edge, W_bond, b_bond,
           W1, b1, W2, b2, W3, b3):
    n = x.shape[0]
    e = edge_attr.shape[0]
    d = emb_atom_w.shape[1]
    rd = W1.shape[1]
    f32 = jnp.float32
    nodeT = jax.ShapeDtypeStruct((d, n), f32)

    x2 = x.reshape(1, n).astype(jnp.int32)
    ea2 = edge_attr[:n].reshape(1, n).astype(jnp.int32)
    ei = edge_index.astype(jnp.int32)
    batch2 = batch.reshape(1, n).astype(jnp.int32)
    bb_col = b_bond.reshape(d, 1)
    bsum_col = (b_msg + b_edge).reshape(d, 1)

    aT0, A1T, Bm1T, packed2 = pl.pallas_call(
        _tc0_body,
        out_shape=(nodeT, nodeT, nodeT,
                   jax.ShapeDtypeStruct((1, e), jnp.int32)),
    )(x2, ea2, ei, emb_atom_w, emb_bond_w, W_bond)
    packed = packed2.reshape(e)

    fill = pl.pallas_call(
        _fill_body,
        grid=(32,),
        out_shape=jax.ShapeDtypeStruct((e, d), f32),
        out_specs=pl.BlockSpec((e // 32, d), lambda i: (i, 0)),
    )()

    edge_pass_deg = _make_edge_pass(n, e, d, True)
    edge_pass = _make_edge_pass(n, e, d, False)
    head_pass_deg = _make_head_pass(n, d, True)
    head_pass = _make_head_pass(n, d, False)

    maxB1T, sumA1T, deg = edge_pass_deg(packed, Bm1T.reshape(d * n),
                                        aT0.reshape(d * n))
    maxB1T = maxB1T.reshape(d, n)
    sumA1T = sumA1T.reshape(d, n)
    deg_row = deg.reshape(1, n)
    nb1T, A2T, Bm2T = pl.pallas_call(
        functools.partial(_tc_nb_body, True),
        out_shape=(nodeT, nodeT, nodeT),
    )(A1T, maxB1T, deg_row, bb_col, W_bond)
    sumb1T, degs = head_pass_deg(packed, nb1T.reshape(d * n))
    sumb1T = sumb1T.reshape(d, n)
    degs_row = degs.reshape(1, n)
    a1T = pl.pallas_call(
        _tc2_body,
        out_shape=nodeT,
    )(sumA1T, sumb1T, deg_row, degs_row, aT0, W_msg, W_edge, bsum_col)

    maxB2T, sumA2T = edge_pass(packed, Bm2T.reshape(d * n),
                               a1T.reshape(d * n))
    maxB2T = maxB2T.reshape(d, n)
    sumA2T = sumA2T.reshape(d, n)
    nb2T, out_bond = pl.pallas_call(
        _tc3_body,
        grid=(1,),
        out_shape=(nodeT, jax.ShapeDtypeStruct((e, d), f32)),
        in_specs=[pl.BlockSpec((d, n), lambda i: (0, 0)),
                  pl.BlockSpec((d, n), lambda i: (0, 0)),
                  pl.BlockSpec((1, n), lambda i: (0, 0)),
                  pl.BlockSpec((d, 1), lambda i: (0, 0)),
                  pl.BlockSpec((n, d), lambda i: (0, 0))],
        out_specs=(pl.BlockSpec((d, n), lambda i: (0, 0)),
                   pl.BlockSpec((n, d), lambda i: (0, 0))),
        input_output_aliases={4: 1},
    )(A2T, maxB2T, deg_row, bb_col, fill)
    sumb2T, = head_pass(packed, nb2T.reshape(d * n))
    sumb2T = sumb2T.reshape(d, n)

    outT, out_atom = pl.pallas_call(
        _tc4_body,
        out_shape=(jax.ShapeDtypeStruct((1, 256), f32),
                   jax.ShapeDtypeStruct((n, d), f32)),
    )(sumA2T, sumb2T, deg_row, degs_row, a1T, W_msg, W_edge, bsum_col,
      batch2, W1, b1.reshape(rd, 1), W2, b2.reshape(rd, 1), W3,
      b3.reshape(1, 1))
    out = outT.reshape(256, 1)
    return (out, out_atom, out_bond)


# speedup vs baseline: 3.2212x; 2.3218x over previous
import functools

import jax
import jax.numpy as jnp
from jax import lax
from jax.experimental import pallas as pl
from jax.experimental.pallas import tpu as pltpu
from jax.experimental.pallas import tpu_sc as plsc

LN2 = 0.6931471805599453
NEG = -3.0e38
NT = 32
ECH = 6400


def _sp(v):
    return jnp.maximum(v, 0.0) + jnp.log(1.0 + jnp.exp(-jnp.abs(v)))


def _sc_mesh():
    return plsc.VectorSubcoreMesh(core_axis_name="c", subcore_axis_name="s",
                                  num_cores=2, num_subcores=16)


def _tc0_body(x_ref, ea_ref, ei_ref, embA_ref, embB_ref, Wb_ref,
              aT_ref, A1T_ref, Bm1T_ref, packed_ref):
    nA = embA_ref.shape[0]
    nB = embB_ref.shape[0]
    n = x_ref.shape[1]
    d = embA_ref.shape[1]
    ohA = (lax.broadcasted_iota(jnp.int32, (nA, n), 0) == x_ref[...]
           ).astype(jnp.float32)
    aT_ref[...] = _sp(lax.dot_general(
        embA_ref[...], ohA, (((0,), (0,)), ((), ())),
        preferred_element_type=jnp.float32,
        precision=lax.Precision.HIGHEST))
    ohB = (lax.broadcasted_iota(jnp.int32, (nB, n), 0) == ea_ref[...]
           ).astype(jnp.float32)
    bT = _sp(lax.dot_general(
        embB_ref[...], ohB, (((0,), (0,)), ((), ())),
        preferred_element_type=jnp.float32,
        precision=lax.Precision.HIGHEST))
    Wb = Wb_ref[...]
    W2b = Wb[d:, :]
    Wd = Wb[:d, :] - W2b
    A1T_ref[...] = lax.dot_general(Wd, bT, (((0,), (0,)), ((), ())),
                                   preferred_element_type=jnp.float32,
        precision=lax.Precision.HIGHEST)
    Bm1T_ref[...] = lax.dot_general(W2b, bT, (((0,), (0,)), ((), ())),
                                    preferred_element_type=jnp.float32,
        precision=lax.Precision.HIGHEST)
    packed_ref[...] = ei_ref[1:2, :] * 65536 + ei_ref[0:1, :]


def _tc_nb_body(with_next, AT_ref, maxT_ref, deg_ref, bb_ref, Wb_ref,
                nbT_ref, *next_refs):
    d = AT_ref.shape[0]
    nb = jnp.where(deg_ref[...] > 0.0,
                   _sp(AT_ref[...] + maxT_ref[...] + bb_ref[...]),
                   LN2)
    nbT_ref[...] = nb
    if with_next:
        A2T_ref, Bm2T_ref = next_refs
        Wb = Wb_ref[...]
        W2b = Wb[d:, :]
        Wd = Wb[:d, :] - W2b
        A2T_ref[...] = lax.dot_general(Wd, nb, (((0,), (0,)), ((), ())),
                                       preferred_element_type=jnp.float32,
        precision=lax.Precision.HIGHEST)
        Bm2T_ref[...] = lax.dot_general(W2b, nb, (((0,), (0,)), ((), ())),
                                        preferred_element_type=jnp.float32,
        precision=lax.Precision.HIGHEST)


def _agg2(sumAT, sumbT, deg, degs, Wm, We, bsum):
    sbf = sumbT + LN2 * (deg - degs)
    return (lax.dot_general(Wm, sumAT, (((0,), (0,)), ((), ())),
                            preferred_element_type=jnp.float32,
        precision=lax.Precision.HIGHEST)
            + lax.dot_general(We, sbf, (((0,), (0,)), ((), ())),
                              preferred_element_type=jnp.float32,
        precision=lax.Precision.HIGHEST)
            + deg * bsum)


def _tc2_body(sumAT_ref, sumbT_ref, deg_ref, degs_ref, aprevT_ref,
              Wm_ref, We_ref, bsum_ref, aT_ref):
    agg = _agg2(sumAT_ref[...], sumbT_ref[...], deg_ref[...], degs_ref[...],
                Wm_ref[...], We_ref[...], bsum_ref[...])
    aT_ref[...] = _sp(agg + aprevT_ref[...])


def _tc3_body(AT_ref, maxT_ref, deg_ref, bb_ref, fill_ref,
              nbT_ref, ob_ref):
    nb = jnp.where(deg_ref[...] > 0.0,
                   _sp(AT_ref[...] + maxT_ref[...] + bb_ref[...]),
                   LN2)
    nbT_ref[...] = nb
    ob_ref[...] = jnp.transpose(nb, (1, 0))


def _tc4_body(sumAT_ref, sumbT_ref, deg_ref, degs_ref, aprevT_ref,
              Wm_ref, We_ref, bsum_ref, batch_ref,
              W1_ref, b1_ref, W2_ref, b2_ref, W3_ref, b3_ref,
              outT_ref, atom_ref):
    n = sumAT_ref.shape[1]
    ng = 256
    agg = _agg2(sumAT_ref[...], sumbT_ref[...], deg_ref[...], degs_ref[...],
                Wm_ref[...], We_ref[...], bsum_ref[...])
    a2 = _sp(agg + aprevT_ref[...])
    atom_ref[...] = jnp.transpose(a2, (1, 0))
    ohg = (lax.broadcasted_iota(jnp.int32, (ng, n), 0) == batch_ref[...]
           ).astype(jnp.float32)
    pooledT = lax.dot_general(a2, ohg, (((1,), (1,)), ((), ())),
                              preferred_element_type=jnp.float32,
        precision=lax.Precision.HIGHEST)
    h = _sp(lax.dot_general(W1_ref[...], pooledT, (((0,), (0,)), ((), ())),
                            preferred_element_type=jnp.float32,
        precision=lax.Precision.HIGHEST)
            + b1_ref[...])
    h = _sp(lax.dot_general(W2_ref[...], h, (((0,), (0,)), ((), ())),
                            preferred_element_type=jnp.float32,
        precision=lax.Precision.HIGHEST)
            + b2_ref[...])
    outT_ref[...] = lax.dot_general(W3_ref[...], h, (((0,), (0,)), ((), ())),
                                    preferred_element_type=jnp.float32,
        precision=lax.Precision.HIGHEST) \
        + b3_ref[...]


def _fill_body(ob_ref):
    ob_ref[...] = jnp.full(ob_ref.shape, LN2, jnp.float32)


def _make_edge_pass(n_nodes, n_edges, d, with_deg):
    cpt = d // NT
    outs = [jax.ShapeDtypeStruct((d * n_nodes,), jnp.float32),
            jax.ShapeDtypeStruct((d * n_nodes,), jnp.float32)]
    scratch = [pltpu.VMEM((cpt * n_nodes,), jnp.float32),
               pltpu.VMEM((cpt * n_nodes,), jnp.float32),
               pltpu.VMEM((ECH,), jnp.int32)]
    if with_deg:
        outs.append(jax.ShapeDtypeStruct((n_nodes,), jnp.float32))
        scratch.append(pltpu.VMEM((n_nodes,), jnp.float32))

    @functools.partial(
        pl.kernel, out_type=tuple(outs), mesh=_sc_mesh(),
        scratch_types=scratch,
        compiler_params=pltpu.CompilerParams(needs_layout_passes=False))
    def edge_pass(packed, bmT, aT, maxT, sumT, *rest):
        if with_deg:
            degO, tin, tout, ebuf, degv = rest
        else:
            tin, tout, ebuf = rest
        cid = lax.axis_index("c")
        sid = lax.axis_index("s")
        wid = sid * 2 + cid
        base = wid * cpt * n_nodes
        offs = [jnp.full((16,), c * n_nodes, jnp.int32) for c in range(cpt)]
        ones16 = jnp.full((16,), 1.0, jnp.float32)

        def zero_tbl(tbl, val):
            nv = tbl.shape[0] // 16

            @functools.partial(plsc.parallel_loop, 0, nv, unroll=8)
            def _(j):
                tbl[pl.ds(j * 16, 16)] = jnp.full((16,), val, jnp.float32)

        def edge_loop(per_group, parallel):
            def chunk_body(ch, _):
                pltpu.sync_copy(packed.at[pl.ds(ch * ECH, ECH)], ebuf)

                def grp_body(g):
                    p = ebuf[pl.ds(g * 16, 16)]
                    dct = lax.shift_right_logical(p, 16)
                    srcv = lax.bitwise_and(p, 65535)
                    per_group(dct, srcv)

                if parallel:
                    @functools.partial(plsc.parallel_loop, 0, ECH // 16,
                                       unroll=4)
                    def _(g):
                        grp_body(g)
                else:
                    def grp(g, _):
                        grp_body(g)
                        return 0
                    lax.fori_loop(0, ECH // 16, grp, 0)
                return 0
            lax.fori_loop(0, n_edges // ECH, chunk_body, 0)

        pltpu.sync_copy(bmT.at[pl.ds(base, cpt * n_nodes)], tin)
        zero_tbl(tout, NEG)
        if with_deg:
            @functools.partial(plsc.parallel_loop, 0, n_nodes // 16, unroll=8)
            def _(j):
                degv[pl.ds(j * 16, 16)] = jnp.zeros((16,), jnp.float32)

        def max_group(dct, srcv):
            gs = [plsc.load_gather(tin, [offs[c] + srcv]) for c in range(cpt)]
            if with_deg:
                plsc.addupdate_scatter(degv, [dct], ones16)
            dis = [offs[c] + dct for c in range(cpt)]
            _, lastm = plsc.scan_count(dct)
            cur = tuple(plsc.load_gather(tout, [dis[c]])
                        for c in range(cpt))
            for c in range(cpt):
                plsc.store_scatter(tout, [dis[c]], gs[c],
                                   mask=gs[c] > cur[c])

            @pl.when(jnp.logical_not(jnp.all(lastm)))
            def _():
                def cond(cu):
                    acc = jnp.any(gs[0] > cu[0])
                    for c in range(1, cpt):
                        acc = jnp.logical_or(acc, jnp.any(gs[c] > cu[c]))
                    return acc

                def bodyw(cu):
                    for c in range(cpt):
                        plsc.store_scatter(tout, [dis[c]], gs[c],
                                           mask=gs[c] > cu[c])
                    return tuple(plsc.load_gather(tout, [dis[c]])
                                 for c in range(cpt))
                lax.while_loop(
                    cond, bodyw,
                    tuple(plsc.load_gather(tout, [dis[c]])
                          for c in range(cpt)))

        edge_loop(max_group, False)
        pltpu.sync_copy(tout, maxT.at[pl.ds(base, cpt * n_nodes)])
        if with_deg:
            @pl.when(wid == 0)
            def _():
                pltpu.sync_copy(degv, degO)

        pltpu.sync_copy(aT.at[pl.ds(base, cpt * n_nodes)], tin)
        zero_tbl(tout, 0.0)

        def sum_group(dct, srcv):
            for c in range(cpt):
                gv = plsc.load_gather(tin, [offs[c] + srcv])
                plsc.addupdate_scatter(tout, [offs[c] + dct], gv)

        edge_loop(sum_group, True)
        pltpu.sync_copy(tout, sumT.at[pl.ds(base, cpt * n_nodes)])

    return edge_pass


def _make_head_pass(n_nodes, d, with_deg):
    cpt = d // NT
    outs = [jax.ShapeDtypeStruct((d * n_nodes,), jnp.float32)]
    scratch = [pltpu.VMEM((cpt * n_nodes,), jnp.float32),
               pltpu.VMEM((cpt * n_nodes,), jnp.float32),
               pltpu.VMEM((n_nodes,), jnp.int32)]
    if with_deg:
        outs.append(jax.ShapeDtypeStruct((n_nodes,), jnp.float32))
        scratch.append(pltpu.VMEM((n_nodes,), jnp.float32))

    @functools.partial(
        pl.kernel, out_type=tuple(outs), mesh=_sc_mesh(),
        scratch_types=scratch,
        compiler_params=pltpu.CompilerParams(needs_layout_passes=False))
    def head_pass(packed, nbT, sumT, *rest):
        if with_deg:
            degO, tin, tout, ebuf, degv = rest
        else:
            tin, tout, ebuf = rest
        cid = lax.axis_index("c")
        sid = lax.axis_index("s")
        wid = sid * 2 + cid
        base = wid * cpt * n_nodes
        offs = [jnp.full((16,), c * n_nodes, jnp.int32) for c in range(cpt)]
        ones16 = jnp.full((16,), 1.0, jnp.float32)

        pltpu.sync_copy(nbT.at[pl.ds(base, cpt * n_nodes)], tin)
        pltpu.sync_copy(packed.at[pl.ds(0, n_nodes)], ebuf)

        @functools.partial(plsc.parallel_loop, 0, cpt * n_nodes // 16,
                           unroll=8)
        def _(j):
            tout[pl.ds(j * 16, 16)] = jnp.zeros((16,), jnp.float32)
        if with_deg:
            @functools.partial(plsc.parallel_loop, 0, n_nodes // 16, unroll=8)
            def _(j):
                degv[pl.ds(j * 16, 16)] = jnp.zeros((16,), jnp.float32)

        @functools.partial(plsc.parallel_loop, 0, n_nodes // 16, unroll=4)
        def _(g):
            p = ebuf[pl.ds(g * 16, 16)]
            dct = lax.shift_right_logical(p, 16)
            if with_deg:
                plsc.addupdate_scatter(degv, [dct], ones16)
            for c in range(cpt):
                gv = tin[pl.ds(c * n_nodes + g * 16, 16)]
                plsc.addupdate_scatter(tout, [offs[c] + dct], gv)

        pltpu.sync_copy(tout, sumT.at[pl.ds(base, cpt * n_nodes)])
        if with_deg:
            @pl.when(wid == 0)
            def _():
                pltpu.sync_copy(degv, degO)

    return head_pass


def kernel(x, edge_attr, edge_index, batch, emb_atom_w, emb_bond_w,
           W_msg, b_msg, W_edge, b_edge, W_bond, b_bond,
           W1, b1, W2, b2, W3, b3):
    n = x.shape[0]
    e = edge_attr.shape[0]
    d = emb_atom_w.shape[1]
    rd = W1.shape[1]
    f32 = jnp.float32
    nodeT = jax.ShapeDtypeStruct((d, n), f32)

    x2 = x.reshape(1, n).astype(jnp.int32)
    ea2 = edge_attr[:n].reshape(1, n).astype(jnp.int32)
    ei = edge_index.astype(jnp.int32)
    batch2 = batch.reshape(1, n).astype(jnp.int32)
    bb_col = b_bond.reshape(d, 1)
    bsum_col = (b_msg + b_edge).reshape(d, 1)

    aT0, A1T, Bm1T, packed2 = pl.pallas_call(
        _tc0_body,
        out_shape=(nodeT, nodeT, nodeT,
                   jax.ShapeDtypeStruct((1, e), jnp.int32)),
    )(x2, ea2, ei, emb_atom_w, emb_bond_w, W_bond)
    packed = packed2.reshape(e)

    fill = pl.pallas_call(
        _fill_body,
        grid=(32,),
        out_shape=jax.ShapeDtypeStruct((e, d), f32),
        out_specs=pl.BlockSpec((e // 32, d), lambda i: (i, 0)),
    )()

    edge_pass_deg = _make_edge_pass(n, e, d, True)
    edge_pass = _make_edge_pass(n, e, d, False)
    head_pass_deg = _make_head_pass(n, d, True)
    head_pass = _make_head_pass(n, d, False)

    maxB1T, sumA1T, deg = edge_pass_deg(packed, Bm1T.reshape(d * n),
                                        aT0.reshape(d * n))
    maxB1T = maxB1T.reshape(d, n)
    sumA1T = sumA1T.reshape(d, n)
    deg_row = deg.reshape(1, n)
    nb1T, A2T, Bm2T = pl.pallas_call(
        functools.partial(_tc_nb_body, True),
        out_shape=(nodeT, nodeT, nodeT),
    )(A1T, maxB1T, deg_row, bb_col, W_bond)
    sumb1T, degs = head_pass_deg(packed, nb1T.reshape(d * n))
    sumb1T = sumb1T.reshape(d, n)
    degs_row = degs.reshape(1, n)
    a1T = pl.pallas_call(
        _tc2_body,
        out_shape=nodeT,
    )(sumA1T, sumb1T, deg_row, degs_row, aT0, W_msg, W_edge, bsum_col)

    maxB2T, sumA2T = edge_pass(packed, Bm2T.reshape(d * n),
                               a1T.reshape(d * n))
    maxB2T = maxB2T.reshape(d, n)
    sumA2T = sumA2T.reshape(d, n)
    nb2T, out_bond = pl.pallas_call(
        _tc3_body,
        grid=(1,),
        out_shape=(nodeT, jax.ShapeDtypeStruct((e, d), f32)),
        in_specs=[pl.BlockSpec((d, n), lambda i: (0, 0)),
                  pl.BlockSpec((d, n), lambda i: (0, 0)),
                  pl.BlockSpec((1, n), lambda i: (0, 0)),
                  pl.BlockSpec((d, 1), lambda i: (0, 0)),
                  pl.BlockSpec((n, d), lambda i: (0, 0))],
        out_specs=(pl.BlockSpec((d, n), lambda i: (0, 0)),
                   pl.BlockSpec((n, d), lambda i: (0, 0))),
        input_output_aliases={4: 1},
    )(A2T, maxB2T, deg_row, bb_col, fill)
    sumb2T, = head_pass(packed, nb2T.reshape(d * n))
    sumb2T = sumb2T.reshape(d, n)

    outT, out_atom = pl.pallas_call(
        _tc4_body,
        out_shape=(jax.ShapeDtypeStruct((1, 256), f32),
                   jax.ShapeDtypeStruct((n, d), f32)),
    )(sumA2T, sumb2T, deg_row, degs_row, a1T, W_msg, W_edge, bsum_col,
      batch2, W1, b1.reshape(rd, 1), W2, b2.reshape(rd, 1), W3,
      b3.reshape(1, 1))
    out = outT.reshape(256, 1)
    return (out, out_atom, out_bond)


# speedup vs baseline: 3.2245x; 1.0010x over previous
import functools

import jax
import jax.numpy as jnp
from jax import lax
from jax.experimental import pallas as pl
from jax.experimental.pallas import tpu as pltpu
from jax.experimental.pallas import tpu_sc as plsc

LN2 = 0.6931471805599453
NEG = -3.0e38
NT = 32
ECH = 6400


def _sp(v):
    return jnp.maximum(v, 0.0) + jnp.log(1.0 + jnp.exp(-jnp.abs(v)))


def _sc_mesh():
    return plsc.VectorSubcoreMesh(core_axis_name="c", subcore_axis_name="s",
                                  num_cores=2, num_subcores=16)


def _tc0_body(x_ref, ea_ref, ei_ref, embA_ref, embB_ref, Wb_ref,
              aT_ref, A1T_ref, Bm1T_ref, packed_ref):
    nA = embA_ref.shape[0]
    nB = embB_ref.shape[0]
    n = x_ref.shape[1]
    d = embA_ref.shape[1]
    ohA = (lax.broadcasted_iota(jnp.int32, (nA, n), 0) == x_ref[...]
           ).astype(jnp.float32)
    aT_ref[...] = _sp(lax.dot_general(
        embA_ref[...], ohA, (((0,), (0,)), ((), ())),
        preferred_element_type=jnp.float32,
        precision=lax.Precision.HIGHEST))
    ohB = (lax.broadcasted_iota(jnp.int32, (nB, n), 0) == ea_ref[...]
           ).astype(jnp.float32)
    bT = _sp(lax.dot_general(
        embB_ref[...], ohB, (((0,), (0,)), ((), ())),
        preferred_element_type=jnp.float32,
        precision=lax.Precision.HIGHEST))
    Wb = Wb_ref[...]
    W2b = Wb[d:, :]
    Wd = Wb[:d, :] - W2b
    A1T_ref[...] = lax.dot_general(Wd, bT, (((0,), (0,)), ((), ())),
                                   preferred_element_type=jnp.float32,
        precision=lax.Precision.HIGHEST)
    Bm1T_ref[...] = lax.dot_general(W2b, bT, (((0,), (0,)), ((), ())),
                                    preferred_element_type=jnp.float32,
        precision=lax.Precision.HIGHEST)
    packed_ref[...] = ei_ref[1:2, :] * 65536 + ei_ref[0:1, :]


def _tc_nb_body(with_next, AT_ref, maxT_ref, deg_ref, bb_ref, Wb_ref,
                nbT_ref, *next_refs):
    d = AT_ref.shape[0]
    nb = jnp.where(deg_ref[...] > 0.0,
                   _sp(AT_ref[...] + maxT_ref[...] + bb_ref[...]),
                   LN2)
    nbT_ref[...] = nb
    if with_next:
        A2T_ref, Bm2T_ref = next_refs
        Wb = Wb_ref[...]
        W2b = Wb[d:, :]
        Wd = Wb[:d, :] - W2b
        A2T_ref[...] = lax.dot_general(Wd, nb, (((0,), (0,)), ((), ())),
                                       preferred_element_type=jnp.float32,
        precision=lax.Precision.HIGHEST)
        Bm2T_ref[...] = lax.dot_general(W2b, nb, (((0,), (0,)), ((), ())),
                                        preferred_element_type=jnp.float32,
        precision=lax.Precision.HIGHEST)


def _agg2(sumAT, sumbT, deg, degs, Wm, We, bsum):
    sbf = sumbT + LN2 * (deg - degs)
    return (lax.dot_general(Wm, sumAT, (((0,), (0,)), ((), ())),
                            preferred_element_type=jnp.float32,
        precision=lax.Precision.HIGHEST)
            + lax.dot_general(We, sbf, (((0,), (0,)), ((), ())),
                              preferred_element_type=jnp.float32,
        precision=lax.Precision.HIGHEST)
            + deg * bsum)


def _tc2_body(sumAT_ref, sumbT_ref, deg_ref, degs_ref, aprevT_ref,
              Wm_ref, We_ref, bsum_ref, aT_ref):
    agg = _agg2(sumAT_ref[...], sumbT_ref[...], deg_ref[...], degs_ref[...],
                Wm_ref[...], We_ref[...], bsum_ref[...])
    aT_ref[...] = _sp(agg + aprevT_ref[...])


def _tc3_body(AT_ref, maxT_ref, deg_ref, bb_ref, fill_ref,
              nbT_ref, ob_ref):
    nb = jnp.where(deg_ref[...] > 0.0,
                   _sp(AT_ref[...] + maxT_ref[...] + bb_ref[...]),
                   LN2)
    nbT_ref[...] = nb
    ob_ref[...] = jnp.transpose(nb, (1, 0))


def _tc4_body(sumAT_ref, sumbT_ref, deg_ref, degs_ref, aprevT_ref,
              Wm_ref, We_ref, bsum_ref, batch_ref,
              W1_ref, b1_ref, W2_ref, b2_ref, W3_ref, b3_ref,
              outT_ref, atom_ref):
    n = sumAT_ref.shape[1]
    ng = 256
    agg = _agg2(sumAT_ref[...], sumbT_ref[...], deg_ref[...], degs_ref[...],
                Wm_ref[...], We_ref[...], bsum_ref[...])
    a2 = _sp(agg + aprevT_ref[...])
    atom_ref[...] = jnp.transpose(a2, (1, 0))
    ohg = (lax.broadcasted_iota(jnp.int32, (ng, n), 0) == batch_ref[...]
           ).astype(jnp.float32)
    pooledT = lax.dot_general(a2, ohg, (((1,), (1,)), ((), ())),
                              preferred_element_type=jnp.float32,
        precision=lax.Precision.HIGHEST)
    h = _sp(lax.dot_general(W1_ref[...], pooledT, (((0,), (0,)), ((), ())),
                            preferred_element_type=jnp.float32,
        precision=lax.Precision.HIGHEST)
            + b1_ref[...])
    h = _sp(lax.dot_general(W2_ref[...], h, (((0,), (0,)), ((), ())),
                            preferred_element_type=jnp.float32,
        precision=lax.Precision.HIGHEST)
            + b2_ref[...])
    outT_ref[...] = lax.dot_general(W3_ref[...], h, (((0,), (0,)), ((), ())),
                                    preferred_element_type=jnp.float32,
        precision=lax.Precision.HIGHEST) \
        + b3_ref[...]


def _fill_body(ob_ref):
    ob_ref[...] = jnp.full(ob_ref.shape, LN2, jnp.float32)


def _make_edge_pass(n_nodes, n_edges, d, with_deg):
    cpt = d // NT
    outs = [jax.ShapeDtypeStruct((d * n_nodes,), jnp.float32),
            jax.ShapeDtypeStruct((d * n_nodes,), jnp.float32)]
    scratch = [pltpu.VMEM((cpt * n_nodes,), jnp.float32),
               pltpu.VMEM((cpt * n_nodes,), jnp.float32),
               pltpu.VMEM((ECH,), jnp.int32)]
    if with_deg:
        outs.append(jax.ShapeDtypeStruct((n_nodes,), jnp.float32))
        scratch.append(pltpu.VMEM((n_nodes,), jnp.float32))

    @functools.partial(
        pl.kernel, out_type=tuple(outs), mesh=_sc_mesh(),
        scratch_types=scratch,
        compiler_params=pltpu.CompilerParams(needs_layout_passes=False))
    def edge_pass(packed, bmT, aT, maxT, sumT, *rest):
        if with_deg:
            degO, tin, tout, ebuf, degv = rest
        else:
            tin, tout, ebuf = rest
        cid = lax.axis_index("c")
        sid = lax.axis_index("s")
        wid = sid * 2 + cid
        base = wid * cpt * n_nodes
        offs = [jnp.full((16,), c * n_nodes, jnp.int32) for c in range(cpt)]
        ones16 = jnp.full((16,), 1.0, jnp.float32)

        def zero_tbl(tbl, val):
            nv = tbl.shape[0] // 16

            @functools.partial(plsc.parallel_loop, 0, nv, unroll=4)
            def _(j):
                tbl[pl.ds(j * 16, 16)] = jnp.full((16,), val, jnp.float32)

        def edge_loop(per_group, parallel):
            def chunk_body(ch, _):
                pltpu.sync_copy(packed.at[pl.ds(ch * ECH, ECH)], ebuf)

                def grp_body(g):
                    p = ebuf[pl.ds(g * 16, 16)]
                    dct = lax.shift_right_logical(p, 16)
                    srcv = lax.bitwise_and(p, 65535)
                    per_group(dct, srcv)

                if parallel:
                    @functools.partial(plsc.parallel_loop, 0, ECH // 16,
                                       unroll=4)
                    def _(g):
                        grp_body(g)
                else:
                    def grp(g, _):
                        grp_body(g)
                        return 0
                    lax.fori_loop(0, ECH // 16, grp, 0)
                return 0
            lax.fori_loop(0, n_edges // ECH, chunk_body, 0)

        pltpu.sync_copy(bmT.at[pl.ds(base, cpt * n_nodes)], tin)
        zero_tbl(tout, NEG)
        if with_deg:
            @functools.partial(plsc.parallel_loop, 0, n_nodes // 16, unroll=5)
            def _(j):
                degv[pl.ds(j * 16, 16)] = jnp.zeros((16,), jnp.float32)

        def max_group(dct, srcv):
            gs = [plsc.load_gather(tin, [offs[c] + srcv]) for c in range(cpt)]
            dis = [offs[c] + dct for c in range(cpt)]
            _, lastm = plsc.scan_count(dct)
            cur = tuple(plsc.load_gather(tout, [dis[c]])
                        for c in range(cpt))
            for c in range(cpt):
                plsc.store_scatter(tout, [dis[c]], gs[c],
                                   mask=gs[c] > cur[c])

            @pl.when(jnp.logical_not(jnp.all(lastm)))
            def _():
                def cond(cu):
                    acc = jnp.any(gs[0] > cu[0])
                    for c in range(1, cpt):
                        acc = jnp.logical_or(acc, jnp.any(gs[c] > cu[c]))
                    return acc

                def bodyw(cu):
                    for c in range(cpt):
                        plsc.store_scatter(tout, [dis[c]], gs[c],
                                           mask=gs[c] > cu[c])
                    return tuple(plsc.load_gather(tout, [dis[c]])
                                 for c in range(cpt))
                lax.while_loop(
                    cond, bodyw,
                    tuple(plsc.load_gather(tout, [dis[c]])
                          for c in range(cpt)))

        edge_loop(max_group, False)
        pltpu.sync_copy(tout, maxT.at[pl.ds(base, cpt * n_nodes)])
        if with_deg:
            @pl.when(wid == 0)
            def _():
                pltpu.sync_copy(degv, degO)

        pltpu.sync_copy(aT.at[pl.ds(base, cpt * n_nodes)], tin)
        zero_tbl(tout, 0.0)

        def sum_group(dct, srcv):
            if with_deg:
                plsc.addupdate_scatter(degv, [dct], ones16)
            for c in range(cpt):
                gv = plsc.load_gather(tin, [offs[c] + srcv])
                plsc.addupdate_scatter(tout, [offs[c] + dct], gv)

        edge_loop(sum_group, True)
        pltpu.sync_copy(tout, sumT.at[pl.ds(base, cpt * n_nodes)])

    return edge_pass


def _make_head_pass(n_nodes, d, with_deg):
    cpt = d // NT
    outs = [jax.ShapeDtypeStruct((d * n_nodes,), jnp.float32)]
    scratch = [pltpu.VMEM((cpt * n_nodes,), jnp.float32),
               pltpu.VMEM((cpt * n_nodes,), jnp.float32),
               pltpu.VMEM((n_nodes,), jnp.int32)]
    if with_deg:
        outs.append(jax.ShapeDtypeStruct((n_nodes,), jnp.float32))
        scratch.append(pltpu.VMEM((n_nodes,), jnp.float32))

    @functools.partial(
        pl.kernel, out_type=tuple(outs), mesh=_sc_mesh(),
        scratch_types=scratch,
        compiler_params=pltpu.CompilerParams(needs_layout_passes=False))
    def head_pass(packed, nbT, sumT, *rest):
        if with_deg:
            degO, tin, tout, ebuf, degv = rest
        else:
            tin, tout, ebuf = rest
        cid = lax.axis_index("c")
        sid = lax.axis_index("s")
        wid = sid * 2 + cid
        base = wid * cpt * n_nodes
        offs = [jnp.full((16,), c * n_nodes, jnp.int32) for c in range(cpt)]
        ones16 = jnp.full((16,), 1.0, jnp.float32)

        pltpu.sync_copy(nbT.at[pl.ds(base, cpt * n_nodes)], tin)
        pltpu.sync_copy(packed.at[pl.ds(0, n_nodes)], ebuf)

        @functools.partial(plsc.parallel_loop, 0, cpt * n_nodes // 16,
                           unroll=4)
        def _(j):
            tout[pl.ds(j * 16, 16)] = jnp.zeros((16,), jnp.float32)
        if with_deg:
            @functools.partial(plsc.parallel_loop, 0, n_nodes // 16, unroll=5)
            def _(j):
                degv[pl.ds(j * 16, 16)] = jnp.zeros((16,), jnp.float32)

        @functools.partial(plsc.parallel_loop, 0, n_nodes // 16, unroll=5)
        def _(g):
            p = ebuf[pl.ds(g * 16, 16)]
            dct = lax.shift_right_logical(p, 16)
            if with_deg:
                plsc.addupdate_scatter(degv, [dct], ones16)
            for c in range(cpt):
                gv = tin[pl.ds(c * n_nodes + g * 16, 16)]
                plsc.addupdate_scatter(tout, [offs[c] + dct], gv)

        pltpu.sync_copy(tout, sumT.at[pl.ds(base, cpt * n_nodes)])
        if with_deg:
            @pl.when(wid == 0)
            def _():
                pltpu.sync_copy(degv, degO)

    return head_pass


def kernel(x, edge_attr, edge_index, batch, emb_atom_w, emb_bond_w,
           W_msg, b_msg, W_edge, b_edge, W_bond, b_bond,
           W1, b1, W2, b2, W3, b3):
    n = x.shape[0]
    e = edge_attr.shape[0]
    d = emb_atom_w.shape[1]
    rd = W1.shape[1]
    f32 = jnp.float32
    nodeT = jax.ShapeDtypeStruct((d, n), f32)

    x2 = x.reshape(1, n).astype(jnp.int32)
    ea2 = edge_attr[:n].reshape(1, n).astype(jnp.int32)
    ei = edge_index.astype(jnp.int32)
    batch2 = batch.reshape(1, n).astype(jnp.int32)
    bb_col = b_bond.reshape(d, 1)
    bsum_col = (b_msg + b_edge).reshape(d, 1)

    aT0, A1T, Bm1T, packed2 = pl.pallas_call(
        _tc0_body,
        out_shape=(nodeT, nodeT, nodeT,
                   jax.ShapeDtypeStruct((1, e), jnp.int32)),
    )(x2, ea2, ei, emb_atom_w, emb_bond_w, W_bond)
    packed = packed2.reshape(e)

    fill = pl.pallas_call(
        _fill_body,
        grid=(32,),
        out_shape=jax.ShapeDtypeStruct((e, d), f32),
        out_specs=pl.BlockSpec((e // 32, d), lambda i: (i, 0)),
    )()

    edge_pass_deg = _make_edge_pass(n, e, d, True)
    edge_pass = _make_edge_pass(n, e, d, False)
    head_pass_deg = _make_head_pass(n, d, True)
    head_pass = _make_head_pass(n, d, False)

    maxB1T, sumA1T, deg = edge_pass_deg(packed, Bm1T.reshape(d * n),
                                        aT0.reshape(d * n))
    maxB1T = maxB1T.reshape(d, n)
    sumA1T = sumA1T.reshape(d, n)
    deg_row = deg.reshape(1, n)
    nb1T, A2T, Bm2T = pl.pallas_call(
        functools.partial(_tc_nb_body, True),
        out_shape=(nodeT, nodeT, nodeT),
    )(A1T, maxB1T, deg_row, bb_col, W_bond)
    sumb1T, degs = head_pass_deg(packed, nb1T.reshape(d * n))
    sumb1T = sumb1T.reshape(d, n)
    degs_row = degs.reshape(1, n)
    a1T = pl.pallas_call(
        _tc2_body,
        out_shape=nodeT,
    )(sumA1T, sumb1T, deg_row, degs_row, aT0, W_msg, W_edge, bsum_col)

    maxB2T, sumA2T = edge_pass(packed, Bm2T.reshape(d * n),
                               a1T.reshape(d * n))
    maxB2T = maxB2T.reshape(d, n)
    sumA2T = sumA2T.reshape(d, n)
    nb2T, out_bond = pl.pallas_call(
        _tc3_body,
        grid=(1,),
        out_shape=(nodeT, jax.ShapeDtypeStruct((e, d), f32)),
        in_specs=[pl.BlockSpec((d, n), lambda i: (0, 0)),
                  pl.BlockSpec((d, n), lambda i: (0, 0)),
                  pl.BlockSpec((1, n), lambda i: (0, 0)),
                  pl.BlockSpec((d, 1), lambda i: (0, 0)),
                  pl.BlockSpec((n, d), lambda i: (0, 0))],
        out_specs=(pl.BlockSpec((d, n), lambda i: (0, 0)),
                   pl.BlockSpec((n, d), lambda i: (0, 0))),
        input_output_aliases={4: 1},
    )(A2T, maxB2T, deg_row, bb_col, fill)
    sumb2T, = head_pass(packed, nb2T.reshape(d * n))
    sumb2T = sumb2T.reshape(d, n)

    outT, out_atom = pl.pallas_call(
        _tc4_body,
        out_shape=(jax.ShapeDtypeStruct((1, 256), f32),
                   jax.ShapeDtypeStruct((n, d), f32)),
    )(sumA2T, sumb2T, deg_row, degs_row, a1T, W_msg, W_edge, bsum_col,
      batch2, W1, b1.reshape(rd, 1), W2, b2.reshape(rd, 1), W3,
      b3.reshape(1, 1))
    out = outT.reshape(256, 1)
    return (out, out_atom, out_bond)
